# Initial kernel scaffold; baseline (speedup 1.0000x reference)
import jax
import jax.numpy as jnp
from jax.experimental import pallas as pl


def kernel(pointcloud, keys, table):
    raise NotImplementedError("write your pallas kernel here")


# speedup vs baseline: 1.6892x; 1.6892x over previous
import jax
import jax.numpy as jnp
from jax.experimental import pallas as pl
from jax.experimental.pallas import tpu as pltpu

B, N, K, D = 4, 2048, 8192, 3
PN = 256


def _nn_body(pts_ref, keys_t_ref, out_ref):
    px = pts_ref[:, 0:1]
    py = pts_ref[:, 1:2]
    pz = pts_ref[:, 2:3]
    kx = keys_t_ref[0:1, :]
    ky = keys_t_ref[1:2, :]
    kz = keys_t_ref[2:3, :]
    dx = px - kx
    dy = py - ky
    dz = pz - kz
    dist = dx * dx + dy * dy + dz * dz
    out_ref[:, :] = jnp.argmin(dist, axis=1, keepdims=True).astype(jnp.int32)


def kernel(pointcloud, keys, table):
    del table
    pts = pointcloud.reshape(B * N, D)
    keys_t = keys.T
    grid = (B * N // PN,)
    ids = pl.pallas_call(
        _nn_body,
        grid=grid,
        in_specs=[
            pl.BlockSpec((PN, D), lambda i: (i, 0)),
            pl.BlockSpec((D, K), lambda i: (0, 0)),
        ],
        out_specs=pl.BlockSpec((PN, 1), lambda i: (i, 0)),
        out_shape=jax.ShapeDtypeStruct((B * N, 1), jnp.int32),
    )(pts, keys_t)
    return (ids.reshape(B, N, 1), pointcloud)
